# baseline (speedup 1.0000x reference)
import jax
import jax.numpy as jnp
from jax import lax
from jax.experimental import pallas as pl
from jax.experimental.pallas import tpu as pltpu
from jax.experimental.pallas import tpu_sc as plsc

TOTAL_TOK = 16384
D_IN = 1024
D_OUT = 1024
NUM_SEQS = 16

NC = 2
NS = 16
NW = NC * NS
ROWS_SC = 4096
ROWS_TC = TOTAL_TOK - ROWS_SC
CHUNK = ROWS_SC // NW
BLK = 16
NBLK = CHUNK // BLK
NBUF = 3
RBLK = 512
LANES = 16
KSL = D_IN // LANES


T_IL = 8
NKG = KSL // T_IL


def _accum_block(buf_v, acc_v, ids_v, j):
    idf = ids_v[pl.ds(j * BLK, LANES)]
    idl = ids_v[pl.ds(j * BLK + BLK - LANES, LANES)]
    seg0 = idf[0]
    segl = idl[LANES - 1]

    @pl.when(seg0 == segl)
    def _fast():
        def _kg(kg, carry):
            sls = [pl.ds(kg * T_IL * LANES + t * LANES, LANES)
                   for t in range(T_IL)]

            def _rows(rr, regs):
                r = rr * 2
                return tuple(
                    (regs[t] + buf_v[r, sls[t]]) + buf_v[r + 1, sls[t]]
                    for t in range(T_IL))

            regs = lax.fori_loop(
                1, BLK // 2, _rows,
                tuple(buf_v[0, sls[t]] + buf_v[1, sls[t]]
                      for t in range(T_IL)))
            for t in range(T_IL):
                acc_v[seg0, sls[t]] = acc_v[seg0, sls[t]] + regs[t]
            return carry

        lax.fori_loop(0, NKG, _kg, 0)

    @pl.when(seg0 != segl)
    def _mixed():
        segs = []
        for g in range(BLK // LANES):
            idv = ids_v[pl.ds(j * BLK + g * LANES, LANES)]
            segs.extend(idv[rl] for rl in range(LANES))

        def _seg(sgv, lo):
            hi = jnp.zeros((), jnp.int32)
            for r in range(BLK):
                hi = hi + (segs[r] <= sgv).astype(jnp.int32)

            @pl.when(hi > lo)
            def _():
                def _row(r, carry):
                    def _kk(kk, c3):
                        for t in range(T_IL):
                            sl = pl.ds(kk * T_IL * LANES + t * LANES, LANES)
                            acc_v[sgv, sl] = acc_v[sgv, sl] + buf_v[r, sl]
                        return c3

                    lax.fori_loop(0, NKG, _kk, 0)
                    return carry

                lax.fori_loop(lo, hi, _row, 0)

            return hi

        lax.fori_loop(0, NUM_SEQS, _seg, jnp.zeros((), jnp.int32))


def _sc_segment_sums_body(flat_hbm, ids_hbm, out_hbm, ids_v, buf0_v, buf1_v,
                          acc_v, sem0, sem1, sem_i):
    c = lax.axis_index("c")
    s = lax.axis_index("s")
    wid = s * NC + c
    base = wid * CHUNK

    bufs = [buf0_v, buf1_v]
    sems = [sem0, sem1]

    def _start_dyn(j, slot):
        pltpu.make_async_copy(
            flat_hbm.at[pl.ds(base + j * BLK, BLK)], bufs[slot],
            sems[slot]).start()

    def _wait(slot):
        pltpu.make_async_copy(
            flat_hbm.at[pl.ds(base, BLK)], bufs[slot], sems[slot]).wait()

    _start_dyn(0, 0)
    _start_dyn(1, 1)
    ids_cp = pltpu.make_async_copy(ids_hbm.at[pl.ds(base, CHUNK)], ids_v,
                                   sem_i)
    ids_cp.start()

    zeros = jnp.zeros((LANES,), jnp.float32)

    def _zero_row(r, carry):
        for k in range(KSL):
            acc_v[r, pl.ds(k * LANES, LANES)] = zeros
        return carry

    lax.fori_loop(0, NUM_SEQS, _zero_row, 0)
    ids_cp.wait()

    def _step(j2, carry):
        j = j2 * 2

        _wait(0)
        _accum_block(buf0_v, acc_v, ids_v, j)

        @pl.when(j + 2 < NBLK)
        def _():
            _start_dyn(j + 2, 0)

        _wait(1)
        _accum_block(buf1_v, acc_v, ids_v, j + 1)

        @pl.when(j + 3 < NBLK)
        def _():
            _start_dyn(j + 3, 1)

        return carry

    lax.fori_loop(0, NBLK // 2, _step, 0)

    pltpu.sync_copy(acc_v, out_hbm.at[wid])


_sc_segment_sums = pl.kernel(
    _sc_segment_sums_body,
    out_type=jax.ShapeDtypeStruct((NW, NUM_SEQS, D_IN), jnp.float32),
    mesh=plsc.VectorSubcoreMesh(core_axis_name="c", subcore_axis_name="s"),
    scratch_types=[
        pltpu.VMEM((CHUNK,), jnp.int32),
        pltpu.VMEM((BLK, D_IN), jnp.float32),
        pltpu.VMEM((BLK, D_IN), jnp.float32),
        pltpu.VMEM((NUM_SEQS, D_IN), jnp.float32),
        pltpu.SemaphoreType.DMA,
        pltpu.SemaphoreType.DMA,
        pltpu.SemaphoreType.DMA,
    ],
)


def _tc_sums_body(flat_ref, ids_ref, out_ref, acc_ref):
    i = pl.program_id(0)
    ids = ids_ref[...]
    seg = lax.broadcasted_iota(jnp.int32, (NUM_SEQS, RBLK), 0)
    onehot = (ids == seg).astype(jnp.float32)
    part = lax.dot_general(onehot, flat_ref[...], (((1,), (0,)), ((), ())),
                           preferred_element_type=jnp.float32)

    @pl.when(i == 0)
    def _():
        acc_ref[...] = jnp.zeros_like(acc_ref)

    acc_ref[...] += part

    @pl.when(i == pl.num_programs(0) - 1)
    def _():
        out_ref[...] = acc_ref[...]


_tc_sums = pl.pallas_call(
    _tc_sums_body,
    grid=(ROWS_TC // RBLK,),
    in_specs=[
        pl.BlockSpec((RBLK, D_IN), lambda i: (ROWS_SC // RBLK + i, 0)),
        pl.BlockSpec((1, RBLK), lambda i: (0, ROWS_SC // RBLK + i)),
    ],
    out_specs=pl.BlockSpec((NUM_SEQS, D_IN), lambda i: (0, 0)),
    out_shape=jax.ShapeDtypeStruct((NUM_SEQS, D_IN), jnp.float32),
    scratch_shapes=[pltpu.VMEM((NUM_SEQS, D_IN), jnp.float32)],
)


def _tc_finish_body(partials_ref, tcsum_ref, ids_ref, w_ref, b_ref, out_ref):
    sums = tcsum_ref[...]
    for w in range(NW):
        sums = sums + partials_ref[w]
    ids = ids_ref[...]
    seg = lax.broadcasted_iota(jnp.int32, (NUM_SEQS, TOTAL_TOK), 0)
    onehot = (ids == seg).astype(jnp.float32)
    counts = jnp.sum(onehot, axis=1, keepdims=True)
    avg = sums / jnp.maximum(counts, 1.0)
    out = lax.dot_general(avg, w_ref[...], (((1,), (1,)), ((), ())),
                          preferred_element_type=jnp.float32)
    out_ref[...] = out + b_ref[...]


_tc_finish = pl.pallas_call(
    _tc_finish_body,
    out_shape=jax.ShapeDtypeStruct((NUM_SEQS, D_OUT), jnp.float32),
)


def kernel(flat, segment_ids, W, b):
    ids32 = segment_ids.astype(jnp.int32)
    ids_row = ids32.reshape(1, TOTAL_TOK)
    partials = _sc_segment_sums(flat, ids32)
    tcsum = _tc_sums(flat, ids_row)
    return _tc_finish(partials, tcsum, ids_row, W, b.reshape(1, D_OUT))


# speedup vs baseline: 1.0014x; 1.0014x over previous
import jax
import jax.numpy as jnp
from jax import lax
from jax.experimental import pallas as pl
from jax.experimental.pallas import tpu as pltpu
from jax.experimental.pallas import tpu_sc as plsc

TOTAL_TOK = 16384
D_IN = 1024
D_OUT = 1024
NUM_SEQS = 16

NC = 2
NS = 16
NW = NC * NS
ROWS_SC = 4096
ROWS_TC = TOTAL_TOK - ROWS_SC
CHUNK = ROWS_SC // NW
BLK = 16
NBLK = CHUNK // BLK
RBLK = 512
LANES = 16
KSL = D_IN // LANES


T_IL = 8
NKG = KSL // T_IL


def _accum_block(buf_v, acc_v, ids_v, j):
    idf = ids_v[pl.ds(j * BLK, LANES)]
    idl = ids_v[pl.ds(j * BLK + BLK - LANES, LANES)]
    seg0 = idf[0]
    segl = idl[LANES - 1]

    @pl.when(seg0 == segl)
    def _fast():
        def _kg(kg, carry):
            sls = [pl.ds(kg * T_IL * LANES + t * LANES, LANES)
                   for t in range(T_IL)]

            def _rows(rr, regs):
                r = rr * 2
                return tuple(
                    (regs[t] + buf_v[r, sls[t]]) + buf_v[r + 1, sls[t]]
                    for t in range(T_IL))

            regs = lax.fori_loop(
                1, BLK // 2, _rows,
                tuple(buf_v[0, sls[t]] + buf_v[1, sls[t]]
                      for t in range(T_IL)))
            for t in range(T_IL):
                acc_v[seg0, sls[t]] = acc_v[seg0, sls[t]] + regs[t]
            return carry

        lax.fori_loop(0, NKG, _kg, 0)

    @pl.when(seg0 != segl)
    def _mixed():
        segs = []
        for g in range(BLK // LANES):
            idv = ids_v[pl.ds(j * BLK + g * LANES, LANES)]
            segs.extend(idv[rl] for rl in range(LANES))

        def _seg(sgv, lo):
            hi = jnp.zeros((), jnp.int32)
            for r in range(BLK):
                hi = hi + (segs[r] <= sgv).astype(jnp.int32)

            @pl.when(hi > lo)
            def _():
                def _row(r, carry):
                    def _kk(kk, c3):
                        for t in range(T_IL):
                            sl = pl.ds(kk * T_IL * LANES + t * LANES, LANES)
                            acc_v[sgv, sl] = acc_v[sgv, sl] + buf_v[r, sl]
                        return c3

                    lax.fori_loop(0, NKG, _kk, 0)
                    return carry

                lax.fori_loop(lo, hi, _row, 0)

            return hi

        lax.fori_loop(0, NUM_SEQS, _seg, jnp.zeros((), jnp.int32))


def _sc_segment_sums_body(flat_hbm, ids_hbm, out_hbm, ids_v, buf0_v, buf1_v,
                          acc_v, sem0, sem1, sem_i):
    c = lax.axis_index("c")
    s = lax.axis_index("s")
    wid = s * NC + c
    base = wid * CHUNK

    bufs = [buf0_v, buf1_v]
    sems = [sem0, sem1]

    def _start_dyn(j, slot):
        pltpu.make_async_copy(
            flat_hbm.at[pl.ds(base + j * BLK, BLK)], bufs[slot],
            sems[slot]).start()

    def _wait(slot):
        pltpu.make_async_copy(
            flat_hbm.at[pl.ds(base, BLK)], bufs[slot], sems[slot]).wait()

    _start_dyn(0, 0)
    _start_dyn(1, 1)
    ids_cp = pltpu.make_async_copy(ids_hbm.at[pl.ds(base, CHUNK)], ids_v,
                                   sem_i)
    ids_cp.start()

    zeros = jnp.zeros((LANES,), jnp.float32)

    def _zero_row(r, carry):
        for k in range(KSL):
            acc_v[r, pl.ds(k * LANES, LANES)] = zeros
        return carry

    lax.fori_loop(0, NUM_SEQS, _zero_row, 0)
    ids_cp.wait()

    def _step(j2, carry):
        j = j2 * 2

        _wait(0)
        _accum_block(buf0_v, acc_v, ids_v, j)

        @pl.when(j + 2 < NBLK)
        def _():
            _start_dyn(j + 2, 0)

        _wait(1)
        _accum_block(buf1_v, acc_v, ids_v, j + 1)

        @pl.when(j + 3 < NBLK)
        def _():
            _start_dyn(j + 3, 1)

        return carry

    lax.fori_loop(0, NBLK // 2, _step, 0)

    pltpu.sync_copy(acc_v, out_hbm.at[wid])


_sc_segment_sums = pl.kernel(
    _sc_segment_sums_body,
    out_type=jax.ShapeDtypeStruct((NW, NUM_SEQS, D_IN), jnp.float32),
    mesh=plsc.VectorSubcoreMesh(core_axis_name="c", subcore_axis_name="s"),
    scratch_types=[
        pltpu.VMEM((CHUNK,), jnp.int32),
        pltpu.VMEM((BLK, D_IN), jnp.float32),
        pltpu.VMEM((BLK, D_IN), jnp.float32),
        pltpu.VMEM((NUM_SEQS, D_IN), jnp.float32),
        pltpu.SemaphoreType.DMA,
        pltpu.SemaphoreType.DMA,
        pltpu.SemaphoreType.DMA,
    ],
)


def _tc_sums_body(flat_ref, ids_ref, out_ref, acc_ref):
    i = pl.program_id(0)
    ids = ids_ref[...]
    seg = lax.broadcasted_iota(jnp.int32, (NUM_SEQS, RBLK), 0)
    onehot = (ids == seg).astype(jnp.float32)
    part = lax.dot_general(onehot, flat_ref[...], (((1,), (0,)), ((), ())),
                           preferred_element_type=jnp.float32)

    @pl.when(i == 0)
    def _():
        acc_ref[...] = jnp.zeros_like(acc_ref)

    acc_ref[...] += part

    @pl.when(i == pl.num_programs(0) - 1)
    def _():
        out_ref[...] = acc_ref[...]


_tc_sums = pl.pallas_call(
    _tc_sums_body,
    grid=(ROWS_TC // RBLK,),
    in_specs=[
        pl.BlockSpec((RBLK, D_IN), lambda i: (ROWS_SC // RBLK + i, 0)),
        pl.BlockSpec((1, RBLK), lambda i: (0, ROWS_SC // RBLK + i)),
    ],
    out_specs=pl.BlockSpec((NUM_SEQS, D_IN), lambda i: (0, 0)),
    out_shape=jax.ShapeDtypeStruct((NUM_SEQS, D_IN), jnp.float32),
    scratch_shapes=[pltpu.VMEM((NUM_SEQS, D_IN), jnp.float32)],
)


def _tc_finish_body(partials_ref, tcsum_ref, ids_ref, w_ref, b_ref, out_ref):
    sums = tcsum_ref[...]
    for w in range(NW):
        sums = sums + partials_ref[w]
    ids = ids_ref[...]
    seg = lax.broadcasted_iota(jnp.int32, (NUM_SEQS, TOTAL_TOK), 0)
    onehot = (ids == seg).astype(jnp.float32)
    counts = jnp.sum(onehot, axis=1, keepdims=True)
    avg = sums / jnp.maximum(counts, 1.0)
    out = lax.dot_general(avg, w_ref[...], (((1,), (1,)), ((), ())),
                          preferred_element_type=jnp.float32)
    out_ref[...] = out + b_ref[...]


_tc_finish = pl.pallas_call(
    _tc_finish_body,
    out_shape=jax.ShapeDtypeStruct((NUM_SEQS, D_OUT), jnp.float32),
)


def kernel(flat, segment_ids, W, b):
    ids32 = segment_ids.astype(jnp.int32)
    ids_row = ids32.reshape(1, TOTAL_TOK)
    partials = _sc_segment_sums(flat, ids32)
    tcsum = _tc_sums(flat, ids_row)
    return _tc_finish(partials, tcsum, ids_row, W, b.reshape(1, D_OUT))


# speedup vs baseline: 1.0101x; 1.0087x over previous
import jax
import jax.numpy as jnp
from jax import lax
from jax.experimental import pallas as pl
from jax.experimental.pallas import tpu as pltpu
from jax.experimental.pallas import tpu_sc as plsc

TOTAL_TOK = 16384
D_IN = 1024
D_OUT = 1024
NUM_SEQS = 16

NC = 2
NS = 16
NW = NC * NS
ROWS_SC = 5120
ROWS_TC = TOTAL_TOK - ROWS_SC
CHUNK = ROWS_SC // NW
BLK = 16
NBLK = CHUNK // BLK
RBLK = 512
LANES = 16
KSL = D_IN // LANES


T_IL = 8
NKG = KSL // T_IL


def _accum_block(buf_v, acc_v, ids_v, j):
    idf = ids_v[pl.ds(j * BLK, LANES)]
    idl = ids_v[pl.ds(j * BLK + BLK - LANES, LANES)]
    seg0 = idf[0]
    segl = idl[LANES - 1]

    @pl.when(seg0 == segl)
    def _fast():
        def _kg(kg, carry):
            sls = [pl.ds(kg * T_IL * LANES + t * LANES, LANES)
                   for t in range(T_IL)]

            def _rows(rr, regs):
                r = rr * 2
                return tuple(
                    (regs[t] + buf_v[r, sls[t]]) + buf_v[r + 1, sls[t]]
                    for t in range(T_IL))

            regs = lax.fori_loop(
                1, BLK // 2, _rows,
                tuple(buf_v[0, sls[t]] + buf_v[1, sls[t]]
                      for t in range(T_IL)))
            for t in range(T_IL):
                acc_v[seg0, sls[t]] = acc_v[seg0, sls[t]] + regs[t]
            return carry

        lax.fori_loop(0, NKG, _kg, 0)

    @pl.when(seg0 != segl)
    def _mixed():
        segs = []
        for g in range(BLK // LANES):
            idv = ids_v[pl.ds(j * BLK + g * LANES, LANES)]
            segs.extend(idv[rl] for rl in range(LANES))

        def _seg(sgv, lo):
            hi = jnp.zeros((), jnp.int32)
            for r in range(BLK):
                hi = hi + (segs[r] <= sgv).astype(jnp.int32)

            @pl.when(hi > lo)
            def _():
                def _row(r, carry):
                    def _kk(kk, c3):
                        for t in range(T_IL):
                            sl = pl.ds(kk * T_IL * LANES + t * LANES, LANES)
                            acc_v[sgv, sl] = acc_v[sgv, sl] + buf_v[r, sl]
                        return c3

                    lax.fori_loop(0, NKG, _kk, 0)
                    return carry

                lax.fori_loop(lo, hi, _row, 0)

            return hi

        lax.fori_loop(0, NUM_SEQS, _seg, jnp.zeros((), jnp.int32))


def _sc_segment_sums_body(flat_hbm, ids_hbm, out_hbm, ids_v, buf0_v, buf1_v,
                          acc_v, sem0, sem1, sem_i):
    c = lax.axis_index("c")
    s = lax.axis_index("s")
    wid = s * NC + c
    base = wid * CHUNK

    bufs = [buf0_v, buf1_v]
    sems = [sem0, sem1]

    def _start_dyn(j, slot):
        pltpu.make_async_copy(
            flat_hbm.at[pl.ds(base + j * BLK, BLK)], bufs[slot],
            sems[slot]).start()

    def _wait(slot):
        pltpu.make_async_copy(
            flat_hbm.at[pl.ds(base, BLK)], bufs[slot], sems[slot]).wait()

    _start_dyn(0, 0)
    _start_dyn(1, 1)
    ids_cp = pltpu.make_async_copy(ids_hbm.at[pl.ds(base, CHUNK)], ids_v,
                                   sem_i)
    ids_cp.start()

    zeros = jnp.zeros((LANES,), jnp.float32)

    def _zero_row(r, carry):
        for k in range(KSL):
            acc_v[r, pl.ds(k * LANES, LANES)] = zeros
        return carry

    lax.fori_loop(0, NUM_SEQS, _zero_row, 0)
    ids_cp.wait()

    def _step(j2, carry):
        j = j2 * 2

        _wait(0)
        _accum_block(buf0_v, acc_v, ids_v, j)

        @pl.when(j + 2 < NBLK)
        def _():
            _start_dyn(j + 2, 0)

        _wait(1)
        _accum_block(buf1_v, acc_v, ids_v, j + 1)

        @pl.when(j + 3 < NBLK)
        def _():
            _start_dyn(j + 3, 1)

        return carry

    lax.fori_loop(0, NBLK // 2, _step, 0)

    pltpu.sync_copy(acc_v, out_hbm.at[wid])


_sc_segment_sums = pl.kernel(
    _sc_segment_sums_body,
    out_type=jax.ShapeDtypeStruct((NW, NUM_SEQS, D_IN), jnp.float32),
    mesh=plsc.VectorSubcoreMesh(core_axis_name="c", subcore_axis_name="s"),
    scratch_types=[
        pltpu.VMEM((CHUNK,), jnp.int32),
        pltpu.VMEM((BLK, D_IN), jnp.float32),
        pltpu.VMEM((BLK, D_IN), jnp.float32),
        pltpu.VMEM((NUM_SEQS, D_IN), jnp.float32),
        pltpu.SemaphoreType.DMA,
        pltpu.SemaphoreType.DMA,
        pltpu.SemaphoreType.DMA,
    ],
)


def _tc_sums_body(flat_ref, ids_ref, out_ref, acc_ref):
    i = pl.program_id(0)
    ids = ids_ref[...]
    seg = lax.broadcasted_iota(jnp.int32, (NUM_SEQS, RBLK), 0)
    onehot = (ids == seg).astype(jnp.float32)
    part = lax.dot_general(onehot, flat_ref[...], (((1,), (0,)), ((), ())),
                           preferred_element_type=jnp.float32)

    @pl.when(i == 0)
    def _():
        acc_ref[...] = jnp.zeros_like(acc_ref)

    acc_ref[...] += part

    @pl.when(i == pl.num_programs(0) - 1)
    def _():
        out_ref[...] = acc_ref[...]


_tc_sums = pl.pallas_call(
    _tc_sums_body,
    grid=(ROWS_TC // RBLK,),
    in_specs=[
        pl.BlockSpec((RBLK, D_IN), lambda i: (ROWS_SC // RBLK + i, 0)),
        pl.BlockSpec((1, RBLK), lambda i: (0, ROWS_SC // RBLK + i)),
    ],
    out_specs=pl.BlockSpec((NUM_SEQS, D_IN), lambda i: (0, 0)),
    out_shape=jax.ShapeDtypeStruct((NUM_SEQS, D_IN), jnp.float32),
    scratch_shapes=[pltpu.VMEM((NUM_SEQS, D_IN), jnp.float32)],
)


def _tc_finish_body(partials_ref, tcsum_ref, ids_ref, w_ref, b_ref, out_ref):
    sums = tcsum_ref[...]
    for w in range(NW):
        sums = sums + partials_ref[w]
    ids = ids_ref[...]
    seg = lax.broadcasted_iota(jnp.int32, (NUM_SEQS, TOTAL_TOK), 0)
    onehot = (ids == seg).astype(jnp.float32)
    counts = jnp.sum(onehot, axis=1, keepdims=True)
    avg = sums / jnp.maximum(counts, 1.0)
    out = lax.dot_general(avg, w_ref[...], (((1,), (1,)), ((), ())),
                          preferred_element_type=jnp.float32)
    out_ref[...] = out + b_ref[...]


_tc_finish = pl.pallas_call(
    _tc_finish_body,
    out_shape=jax.ShapeDtypeStruct((NUM_SEQS, D_OUT), jnp.float32),
)


def kernel(flat, segment_ids, W, b):
    ids32 = segment_ids.astype(jnp.int32)
    ids_row = ids32.reshape(1, TOTAL_TOK)
    partials = _sc_segment_sums(flat, ids32)
    tcsum = _tc_sums(flat, ids_row)
    return _tc_finish(partials, tcsum, ids_row, W, b.reshape(1, D_OUT))
